# baseline (speedup 1.0000x reference)
import functools

import jax
import jax.numpy as jnp
from jax import lax
from jax.experimental import pallas as pl
from jax.experimental.pallas import tpu as pltpu
from jax.experimental.pallas import tpu_sc as plsc

_F32 = jnp.float32
_I32 = jnp.int32
_BIG = 3.0e38

_USE_SC = True


def _knn_body(q_ref, kk_ref, o_ref, *, nq, nk, k):
    b = pl.program_id(0)
    q = q_ref[0]
    kk = kk_ref[0]
    d2q = jnp.sum(q * q, axis=1, keepdims=True)
    d2k = jnp.transpose(jnp.sum(kk * kk, axis=1, keepdims=True))
    qb = q.astype(jnp.bfloat16)
    kb = kk.astype(jnp.bfloat16)
    qk = lax.dot_general(qb, kb, (((1,), (1,)), ((), ())),
                         preferred_element_type=_F32)
    dist = d2q + d2k - 2.0 * qk
    iota = lax.broadcasted_iota(_I32, (nq, nk), 1)
    base = b * nk
    for j in range(k):
        m = jnp.min(dist, axis=1, keepdims=True)
        eq = dist == m
        am = jnp.min(jnp.where(eq, iota, jnp.int32(2 ** 30)), axis=1,
                     keepdims=True)
        o_ref[0, :, pl.ds(j, 1)] = am + base
        dist = jnp.where(iota == am, _BIG, dist)


def _knn_pallas(pos_pad, k=16, mq=256):
    B, N, _ = pos_pad.shape
    mq = min(mq, N)
    grid = (B, N // mq)
    return pl.pallas_call(
        functools.partial(_knn_body, nq=mq, nk=N, k=k),
        grid=grid,
        in_specs=[
            pl.BlockSpec((1, mq, 8), lambda b, i: (b, i, 0)),
            pl.BlockSpec((1, N, 8), lambda b, i: (b, 0, 0)),
        ],
        out_specs=pl.BlockSpec((1, mq, k), lambda b, i: (b, i, 0)),
        out_shape=jax.ShapeDtypeStruct((B, N, k), _I32),
    )(pos_pad, pos_pad)


def _fps_body(p_ref, o_ref, dist_ref, *, B, N, m):
    iota = lax.broadcasted_iota(_I32, (1, N), 1)
    iota8 = lax.broadcasted_iota(_I32, (8, N), 1)
    for b in range(B):
        p = p_ref[b]
        p0 = p[:, 0:1]
        dist_ref[pl.ds(b, 1), :] = jnp.sum((p - p0) ** 2, axis=0,
                                           keepdims=True)
        o_ref[b, pl.ds(0, 1), :] = jnp.zeros((1, 128), _I32)

    def step(i, carry):
        d = dist_ref[:, :]
        mx = jnp.max(d, axis=1, keepdims=True)
        am = jnp.min(jnp.where(d == mx, iota8, jnp.int32(2 ** 30)),
                     axis=1, keepdims=True)
        for b in range(B):
            nxt = am[b, 0]
            p = p_ref[b]
            oh = (iota == nxt[None, None]).astype(_F32)
            pn = jnp.sum(p * oh, axis=1, keepdims=True)
            nd = jnp.sum((p - pn) ** 2, axis=0, keepdims=True)
            dist_ref[pl.ds(b, 1), :] = jnp.minimum(d[b:b + 1], nd)
            o_ref[b, pl.ds(i, 1), :] = jnp.broadcast_to(
                nxt[None, None], (1, 128))
        return carry

    lax.fori_loop(1, m, step, 0)


def _fps_pallas_call(posT, m):
    B = posT.shape[0]
    N = posT.shape[2]
    return pl.pallas_call(
        functools.partial(_fps_body, B=B, N=N, m=m),
        in_specs=[pl.BlockSpec((B, 8, N), lambda: (0, 0, 0))],
        out_specs=pl.BlockSpec((B, m, 128), lambda: (0, 0, 0)),
        out_shape=jax.ShapeDtypeStruct((B, m, 128), _I32),
        scratch_shapes=[pltpu.VMEM((8, N), _F32)],
    )(posT)


def _sc_gather_max(Z, idx3, group, R):
    T, D = Z.shape
    NW = 32
    n_ch = idx3.shape[1]
    rows_per_chunk = 128 // group
    r_w = R // NW
    mesh = plsc.VectorSubcoreMesh(core_axis_name="c", subcore_axis_name="s")

    @functools.partial(
        pl.kernel, mesh=mesh,
        out_type=jax.ShapeDtypeStruct((R, D), _F32),
        compiler_params=pltpu.CompilerParams(use_tc_tiling_on_sc=False),
        scratch_types=[
            pltpu.VMEM((n_ch, 128), _I32),
            pltpu.VMEM((128, D), _F32),
            pltpu.VMEM((rows_per_chunk, D), _F32),
            pltpu.SemaphoreType.DMA,
        ],
    )
    def k(z_hbm, idx_hbm, out_hbm, idx_v, rows_v, cout_v, sem):
        wid = lax.axis_index("s") * 2 + lax.axis_index("c")
        pltpu.sync_copy(idx_hbm.at[wid], idx_v)

        def chunk(ch, carry):
            pltpu.async_copy(z_hbm.at[idx_v.at[ch]], rows_v, sem).wait()
            for o in range(rows_per_chunk):
                for c in range(D // 16):
                    acc = rows_v[o * group, pl.ds(c * 16, 16)]
                    for g in range(1, group):
                        acc = jnp.maximum(
                            acc, rows_v[o * group + g, pl.ds(c * 16, 16)])
                    cout_v[o, pl.ds(c * 16, 16)] = acc
            pltpu.sync_copy(
                cout_v,
                out_hbm.at[pl.ds(wid * r_w + ch * rows_per_chunk,
                                 rows_per_chunk)])
            return carry

        lax.fori_loop(0, n_ch, chunk, 0)

    return k(Z, idx3)


def _gather_max(Z, idx_flat, group, R):
    if _USE_SC:
        idx3 = idx_flat.reshape(32, -1, 128)
        return _sc_gather_max(Z, idx3, group, R)
    g = Z[idx_flat].reshape(R, group, Z.shape[1])
    return jnp.max(g, axis=1)


def _gather_rows(Z, idx_flat):
    n = idx_flat.shape[0]
    npad = -(-n // 4096) * 4096
    if npad != n:
        idx_flat = jnp.pad(idx_flat, (0, npad - n))
    out = _gather_max(Z, idx_flat, 1, npad)
    return out[:n]


def _layer_body(g_ref, pos_ref, wf_ref, wp_ref, b_ref, o_ref, *, act):
    pos = pos_ref[...]
    wf = wf_ref[...].astype(jnp.bfloat16)
    wp = wp_ref[...].astype(jnp.bfloat16)
    acc = None
    for j in range(16):
        nb = g_ref[j, :, 16:]
        rel = g_ref[j, :, 0:8] - pos
        hj = jnp.dot(nb.astype(jnp.bfloat16), wf,
                     preferred_element_type=_F32) \
            + jnp.dot(rel.astype(jnp.bfloat16), wp,
                      preferred_element_type=_F32)
        acc = hj if acc is None else jnp.maximum(acc, hj)
    h = acc + b_ref[...]
    if act == "relu":
        h = jnp.maximum(h, 0.0)
    o_ref[:, 0:8] = pos
    o_ref[:, 8:16] = jnp.zeros_like(pos)
    o_ref[:, 16:] = h


def _tc_layer(g16, pos8, Wf, Wp, b, act):
    R = pos8.shape[0]
    Cw = g16.shape[2]
    Dpad = Wf.shape[1]
    Cpad = Cw - 16
    Rblk = min(2048, R)
    return pl.pallas_call(
        functools.partial(_layer_body, act=act),
        grid=(R // Rblk,),
        in_specs=[
            pl.BlockSpec((16, Rblk, Cw), lambda i: (0, i, 0)),
            pl.BlockSpec((Rblk, 8), lambda i: (i, 0)),
            pl.BlockSpec((Cpad, Dpad), lambda i: (0, 0)),
            pl.BlockSpec((8, Dpad), lambda i: (0, 0)),
            pl.BlockSpec((1, Dpad), lambda i: (0, 0)),
        ],
        out_specs=pl.BlockSpec((Rblk, 16 + Dpad), lambda i: (i, 0)),
        out_shape=jax.ShapeDtypeStruct((R, 16 + Dpad), _F32),
    )(g16, pos8, Wf, Wp, b)


def _dec_body(cat_ref, featp_ref, wg_ref, bg_ref, e_ref,
              w0_ref, b0_ref, ws0_ref, bs0_ref,
              w1_ref, b1_ref, ws1_ref, bs1_ref,
              w2_ref, b2_ref, ws2_ref, bs2_ref, o_ref, *, M, R):
    gm = jnp.dot(featp_ref[0].astype(jnp.bfloat16),
                 wg_ref[...].astype(jnp.bfloat16),
                 preferred_element_type=_F32) + bg_ref[...]
    gm = jnp.maximum(gm, 0.0)
    g = jnp.max(gm, axis=0, keepdims=True)

    cat = cat_ref[0]
    x = (cat[:, None, :] + e_ref[...][None, :, :]).reshape(M * R, 16)

    def adain(h, w_ref, b_ref, ws_ref, bs_ref, act):
        h = jnp.dot(h.astype(jnp.bfloat16), w_ref[...].astype(jnp.bfloat16),
                    preferred_element_type=_F32) + b_ref[...]
        mu = jnp.mean(h, axis=0, keepdims=True)
        var = jnp.mean((h - mu) ** 2, axis=0, keepdims=True)
        hn = (h - mu) / jnp.sqrt(var + 1e-5)
        style = jnp.dot(g.astype(jnp.bfloat16),
                        ws_ref[...].astype(jnp.bfloat16),
                        preferred_element_type=_F32) + bs_ref[...]
        C = h.shape[1]
        gamma = style[:, :C]
        beta = style[:, C:]
        h = hn * (1.0 + gamma) + beta
        if act == "relu":
            h = jnp.maximum(h, 0.0)
        return h

    h = adain(x, w0_ref, b0_ref, ws0_ref, bs0_ref, "relu")
    h = adain(h, w1_ref, b1_ref, ws1_ref, bs1_ref, "relu")
    h = adain(h, w2_ref, b2_ref, ws2_ref, bs2_ref, "none")
    o_ref[0] = h


def _decoder(cat, featp, Wg, bg, Egrid, dec_W, dec_b, dec_Ws, dec_bs):
    B, M, _ = cat.shape
    R = Egrid.shape[0]
    full = lambda *s: pl.BlockSpec(s, lambda b: (0,) * len(s))
    batched = lambda *s: pl.BlockSpec((1,) + s, lambda b: (b, 0, 0))
    return pl.pallas_call(
        functools.partial(_dec_body, M=M, R=R),
        grid=(B,),
        in_specs=[
            batched(M, 16), batched(M, 16),
            full(16, 64), full(1, 64), full(R, 16),
            full(16, 64), full(1, 64), full(64, 128), full(1, 128),
            full(64, 64), full(1, 64), full(64, 128), full(1, 128),
            full(64, 8), full(1, 8), full(64, 16), full(1, 16),
        ],
        out_specs=pl.BlockSpec((1, M * R, 8), lambda b: (b, 0, 0)),
        out_shape=jax.ShapeDtypeStruct((B, M * R, 8), _F32),
    )(cat, featp, Wg, bg, Egrid,
      dec_W[0], dec_b[0], dec_Ws[0], dec_bs[0],
      dec_W[1], dec_b[1], dec_Ws[1], dec_bs[1],
      dec_W[2], dec_b[2], dec_Ws[2], dec_bs[2])


def _pad_cols(x, w):
    c = x.shape[-1]
    if c == w:
        return x
    return jnp.pad(x, [(0, 0)] * (x.ndim - 1) + [(0, w - c)])


def _pad_rows(x, w):
    r = x.shape[0]
    if r == w:
        return x
    return jnp.pad(x, [(0, w - r)] + [(0, 0)] * (x.ndim - 1))


def _stage(table, pos8, idx_jflat, Ws, bs, acts, R):
    for l, W in enumerate(Ws):
        Cin = W.shape[0] - 3
        D = W.shape[1]
        Dpad = -(-D // 16) * 16
        Cpad = table.shape[1] - 16
        Wf = _pad_cols(_pad_rows(W[:Cin], Cpad), Dpad)
        Wp = _pad_cols(_pad_rows(W[Cin:], 8), Dpad)
        b = _pad_cols(bs[l][None, :], Dpad)
        g = _gather_rows(table, idx_jflat)
        g16 = g.reshape(16, R, table.shape[1])
        table = _tc_layer(g16, pos8, Wf, Wp, b, acts[l])
    return table


def kernel(pos, feat, enc_W, enc_b, Wg, bg, Egrid, dec_W, dec_b, dec_Ws,
           dec_bs):
    B, N, _ = pos.shape

    def jmaj(idx, R):
        return jnp.transpose(idx.reshape(R, 16)).reshape(-1)

    pos8 = _pad_cols(pos, 8)
    idx = _knn_pallas(pos8)
    R1 = B * N
    pos8f = pos8.reshape(R1, 8)
    table1 = jnp.concatenate(
        [_pad_cols(pos8f, 16), _pad_cols(feat.reshape(R1, 1), 16)], axis=1)
    t1 = _stage(table1, pos8f, jmaj(idx, R1), enc_W[0:3], enc_b[0:3],
                ["relu", "relu", "relu"], R1)

    m1 = N // 4
    posT = jnp.transpose(pos8, (0, 2, 1))
    s1 = _fps_pallas_call(posT, m1)[:, :, 0]
    s1g = (s1 + (jnp.arange(B, dtype=_I32) * N)[:, None]).reshape(-1)
    R2 = B * m1
    rows2 = _gather_rows(t1, s1g)
    pos8_2 = rows2[:, :8]
    pos2 = pos8_2[:, :3].reshape(B, m1, 3)

    idx2 = _knn_pallas(_pad_cols(pos2, 8))
    t2 = _stage(rows2, pos8_2, jmaj(idx2, R2), enc_W[3:7], enc_b[3:7],
                ["relu"] * 4, R2)

    m2 = m1 // 16
    posT2 = jnp.transpose(pos2, (0, 2, 1))
    posT2 = jnp.pad(posT2, ((0, 0), (0, 5), (0, 0)))
    s2 = _fps_pallas_call(posT2, m2)[:, :, 0]
    s2g = (s2 + (jnp.arange(B, dtype=_I32) * m1)[:, None]).reshape(-1)
    R3 = B * m2
    rows3 = _gather_rows(t2, s2g)
    pos8_3 = rows3[:, :8]
    pos3 = pos8_3[:, :3].reshape(B, m2, 3)

    idx3 = _knn_pallas(_pad_cols(pos3, 8))
    t3 = _stage(rows3, pos8_3, jmaj(idx3, R3), enc_W[7:14], enc_b[7:14],
                ["relu"] * 6 + ["none"], R3)
    f3 = t3[:, 16:]

    latent_pos = pos3
    latent_feat = f3[:, :13].reshape(B, m2, 13)

    featp = _pad_cols(f3[:, :13], 16).reshape(B, m2, 16)
    cat = jnp.concatenate([latent_feat, latent_pos], axis=2)
    Wg16 = _pad_rows(Wg, 16)
    dW = [dec_W[0], dec_W[1], _pad_cols(dec_W[2], 8)]
    db = [dec_b[0][None, :], dec_b[1][None, :],
          _pad_cols(dec_b[2][None, :], 8)]
    dWs = [dec_Ws[0], dec_Ws[1], _pad_cols(dec_Ws[2].reshape(64, 2, 3),
                                           8).reshape(64, 16)]
    dbs = [dec_bs[0][None, :], dec_bs[1][None, :],
          _pad_cols(dec_bs[2].reshape(1, 2, 3), 8).reshape(1, 16)]
    dec = _decoder(cat, featp, Wg16, bg[None, :], Egrid, dW, db, dWs, dbs)
    dec = dec[:, :, :3]
    return (latent_pos, latent_feat, dec)


# speedup vs baseline: 1.0003x; 1.0003x over previous
import functools

import jax
import jax.numpy as jnp
from jax import lax
from jax.experimental import pallas as pl
from jax.experimental.pallas import tpu as pltpu
from jax.experimental.pallas import tpu_sc as plsc

_F32 = jnp.float32
_I32 = jnp.int32
_BIG = 3.0e38


def _knn_body(q_ref, kk_ref, o_ref, *, nq, nk, k):
    b = pl.program_id(0)
    q = q_ref[0]
    kk = kk_ref[0]
    d2q = jnp.sum(q * q, axis=1, keepdims=True)
    d2k = jnp.transpose(jnp.sum(kk * kk, axis=1, keepdims=True))
    qb = q.astype(jnp.bfloat16)
    kb = kk.astype(jnp.bfloat16)
    qk = lax.dot_general(qb, kb, (((1,), (1,)), ((), ())),
                         preferred_element_type=_F32)
    dist = d2q + d2k - 2.0 * qk
    iota = lax.broadcasted_iota(_I32, (nq, nk), 1)
    base = b * nk
    for j in range(k):
        m = jnp.min(dist, axis=1, keepdims=True)
        eq = dist == m
        am = jnp.min(jnp.where(eq, iota, jnp.int32(2 ** 30)), axis=1,
                     keepdims=True)
        o_ref[0, :, pl.ds(j, 1)] = am + base
        dist = jnp.where(iota == am, _BIG, dist)


def _knn_pallas(pos_pad, k=16, mq=256):
    B, N, _ = pos_pad.shape
    mq = min(mq, N)
    grid = (B, N // mq)
    return pl.pallas_call(
        functools.partial(_knn_body, nq=mq, nk=N, k=k),
        grid=grid,
        in_specs=[
            pl.BlockSpec((1, mq, 8), lambda b, i: (b, i, 0)),
            pl.BlockSpec((1, N, 8), lambda b, i: (b, 0, 0)),
        ],
        out_specs=pl.BlockSpec((1, mq, k), lambda b, i: (b, i, 0)),
        out_shape=jax.ShapeDtypeStruct((B, N, k), _I32),
    )(pos_pad, pos_pad)


def _fps_body(p_ref, o_ref, dist_ref, *, B, N, m):
    iota = lax.broadcasted_iota(_I32, (1, N), 1)
    iota8 = lax.broadcasted_iota(_I32, (8, N), 1)
    for b in range(B):
        p = p_ref[b]
        p0 = p[:, 0:1]
        dist_ref[pl.ds(b, 1), :] = jnp.sum((p - p0) ** 2, axis=0,
                                           keepdims=True)
        o_ref[b, pl.ds(0, 1), :] = jnp.zeros((1, 128), _I32)

    def step(i, carry):
        d = dist_ref[:, :]
        mx = jnp.max(d, axis=1, keepdims=True)
        am = jnp.min(jnp.where(d == mx, iota8, jnp.int32(2 ** 30)),
                     axis=1, keepdims=True)
        for b in range(B):
            nxt = am[b, 0]
            p = p_ref[b]
            oh = (iota == nxt[None, None]).astype(_F32)
            pn = jnp.sum(p * oh, axis=1, keepdims=True)
            nd = jnp.sum((p - pn) ** 2, axis=0, keepdims=True)
            dist_ref[pl.ds(b, 1), :] = jnp.minimum(d[b:b + 1], nd)
            o_ref[b, pl.ds(i, 1), :] = jnp.broadcast_to(
                nxt[None, None], (1, 128))
        return carry

    lax.fori_loop(1, m, step, 0)


def _fps_pallas_call(posT, m):
    B = posT.shape[0]
    N = posT.shape[2]
    return pl.pallas_call(
        functools.partial(_fps_body, B=B, N=N, m=m),
        in_specs=[pl.BlockSpec((B, 8, N), lambda: (0, 0, 0))],
        out_specs=pl.BlockSpec((B, m, 128), lambda: (0, 0, 0)),
        out_shape=jax.ShapeDtypeStruct((B, m, 128), _I32),
        scratch_shapes=[pltpu.VMEM((8, N), _F32)],
    )(posT)


def _sc_gather_max(Z, idx3, group, R):
    T, D = Z.shape
    NW = 32
    n_ch = idx3.shape[1]
    rows_per_chunk = 128 // group
    r_w = R // NW
    mesh = plsc.VectorSubcoreMesh(core_axis_name="c", subcore_axis_name="s")

    @functools.partial(
        pl.kernel, mesh=mesh,
        out_type=jax.ShapeDtypeStruct((R, D), _F32),
        compiler_params=pltpu.CompilerParams(use_tc_tiling_on_sc=False),
        scratch_types=[
            pltpu.VMEM((n_ch, 128), _I32),
            pltpu.VMEM((128, D), _F32),
            pltpu.VMEM((rows_per_chunk, D), _F32),
            pltpu.SemaphoreType.DMA,
        ],
    )
    def k(z_hbm, idx_hbm, out_hbm, idx_v, rows_v, cout_v, sem):
        wid = lax.axis_index("s") * 2 + lax.axis_index("c")
        pltpu.sync_copy(idx_hbm.at[wid], idx_v)

        def chunk(ch, carry):
            pltpu.async_copy(z_hbm.at[idx_v.at[ch]], rows_v, sem).wait()
            for o in range(rows_per_chunk):
                for c in range(D // 16):
                    acc = rows_v[o * group, pl.ds(c * 16, 16)]
                    for g in range(1, group):
                        acc = jnp.maximum(
                            acc, rows_v[o * group + g, pl.ds(c * 16, 16)])
                    cout_v[o, pl.ds(c * 16, 16)] = acc
            pltpu.sync_copy(
                cout_v,
                out_hbm.at[pl.ds(wid * r_w + ch * rows_per_chunk,
                                 rows_per_chunk)])
            return carry

        lax.fori_loop(0, n_ch, chunk, 0)

    return k(Z, idx3)


def _gather_max(Z, idx_flat, group, R):
    idx3 = idx_flat.reshape(32, -1, 128)
    return _sc_gather_max(Z, idx3, group, R)


def _gather_rows(Z, idx_flat):
    n = idx_flat.shape[0]
    npad = -(-n // 4096) * 4096
    if npad != n:
        idx_flat = jnp.pad(idx_flat, (0, npad - n))
    out = _gather_max(Z, idx_flat, 1, npad)
    return out[:n]


def _layer_body(g_ref, pos_ref, wf_ref, wp_ref, b_ref, o_ref, *, act):
    pos = pos_ref[...]
    wf = wf_ref[...].astype(jnp.bfloat16)
    wp = wp_ref[...].astype(jnp.bfloat16)
    acc = None
    for j in range(16):
        nb = g_ref[j, :, 16:]
        rel = g_ref[j, :, 0:8] - pos
        hj = jnp.dot(nb.astype(jnp.bfloat16), wf,
                     preferred_element_type=_F32) \
            + jnp.dot(rel.astype(jnp.bfloat16), wp,
                      preferred_element_type=_F32)
        acc = hj if acc is None else jnp.maximum(acc, hj)
    h = acc + b_ref[...]
    if act == "relu":
        h = jnp.maximum(h, 0.0)
    o_ref[:, 0:8] = pos
    o_ref[:, 8:16] = jnp.zeros_like(pos)
    o_ref[:, 16:] = h


def _tc_layer(g16, pos8, Wf, Wp, b, act):
    R = pos8.shape[0]
    Cw = g16.shape[2]
    Dpad = Wf.shape[1]
    Cpad = Cw - 16
    Rblk = min(2048, R)
    return pl.pallas_call(
        functools.partial(_layer_body, act=act),
        grid=(R // Rblk,),
        in_specs=[
            pl.BlockSpec((16, Rblk, Cw), lambda i: (0, i, 0)),
            pl.BlockSpec((Rblk, 8), lambda i: (i, 0)),
            pl.BlockSpec((Cpad, Dpad), lambda i: (0, 0)),
            pl.BlockSpec((8, Dpad), lambda i: (0, 0)),
            pl.BlockSpec((1, Dpad), lambda i: (0, 0)),
        ],
        out_specs=pl.BlockSpec((Rblk, 16 + Dpad), lambda i: (i, 0)),
        out_shape=jax.ShapeDtypeStruct((R, 16 + Dpad), _F32),
    )(g16, pos8, Wf, Wp, b)


def _dec_body(cat_ref, featp_ref, wg_ref, bg_ref, e_ref,
              w0_ref, b0_ref, ws0_ref, bs0_ref,
              w1_ref, b1_ref, ws1_ref, bs1_ref,
              w2_ref, b2_ref, ws2_ref, bs2_ref, o_ref, *, M, R):
    gm = jnp.dot(featp_ref[0].astype(jnp.bfloat16),
                 wg_ref[...].astype(jnp.bfloat16),
                 preferred_element_type=_F32) + bg_ref[...]
    gm = jnp.maximum(gm, 0.0)
    g = jnp.max(gm, axis=0, keepdims=True)

    cat = cat_ref[0]
    x = (cat[:, None, :] + e_ref[...][None, :, :]).reshape(M * R, 16)

    def adain(h, w_ref, b_ref, ws_ref, bs_ref, act):
        h = jnp.dot(h.astype(jnp.bfloat16), w_ref[...].astype(jnp.bfloat16),
                    preferred_element_type=_F32) + b_ref[...]
        mu = jnp.mean(h, axis=0, keepdims=True)
        var = jnp.mean((h - mu) ** 2, axis=0, keepdims=True)
        hn = (h - mu) / jnp.sqrt(var + 1e-5)
        style = jnp.dot(g.astype(jnp.bfloat16),
                        ws_ref[...].astype(jnp.bfloat16),
                        preferred_element_type=_F32) + bs_ref[...]
        C = h.shape[1]
        gamma = style[:, :C]
        beta = style[:, C:]
        h = hn * (1.0 + gamma) + beta
        if act == "relu":
            h = jnp.maximum(h, 0.0)
        return h

    h = adain(x, w0_ref, b0_ref, ws0_ref, bs0_ref, "relu")
    h = adain(h, w1_ref, b1_ref, ws1_ref, bs1_ref, "relu")
    h = adain(h, w2_ref, b2_ref, ws2_ref, bs2_ref, "none")
    o_ref[0] = h


def _decoder(cat, featp, Wg, bg, Egrid, dec_W, dec_b, dec_Ws, dec_bs):
    B, M, _ = cat.shape
    R = Egrid.shape[0]
    full = lambda *s: pl.BlockSpec(s, lambda b: (0,) * len(s))
    batched = lambda *s: pl.BlockSpec((1,) + s, lambda b: (b, 0, 0))
    return pl.pallas_call(
        functools.partial(_dec_body, M=M, R=R),
        grid=(B,),
        in_specs=[
            batched(M, 16), batched(M, 16),
            full(16, 64), full(1, 64), full(R, 16),
            full(16, 64), full(1, 64), full(64, 128), full(1, 128),
            full(64, 64), full(1, 64), full(64, 128), full(1, 128),
            full(64, 8), full(1, 8), full(64, 16), full(1, 16),
        ],
        out_specs=pl.BlockSpec((1, M * R, 8), lambda b: (b, 0, 0)),
        out_shape=jax.ShapeDtypeStruct((B, M * R, 8), _F32),
    )(cat, featp, Wg, bg, Egrid,
      dec_W[0], dec_b[0], dec_Ws[0], dec_bs[0],
      dec_W[1], dec_b[1], dec_Ws[1], dec_bs[1],
      dec_W[2], dec_b[2], dec_Ws[2], dec_bs[2])


def _pad_cols(x, w):
    c = x.shape[-1]
    if c == w:
        return x
    return jnp.pad(x, [(0, 0)] * (x.ndim - 1) + [(0, w - c)])


def _pad_rows(x, w):
    r = x.shape[0]
    if r == w:
        return x
    return jnp.pad(x, [(0, w - r)] + [(0, 0)] * (x.ndim - 1))


def _stage(table, pos8, idx_jflat, Ws, bs, acts, R):
    for l, W in enumerate(Ws):
        Cin = W.shape[0] - 3
        D = W.shape[1]
        Dpad = -(-D // 16) * 16
        Cpad = table.shape[1] - 16
        Wf = _pad_cols(_pad_rows(W[:Cin], Cpad), Dpad)
        Wp = _pad_cols(_pad_rows(W[Cin:], 8), Dpad)
        b = _pad_cols(bs[l][None, :], Dpad)
        g = _gather_rows(table, idx_jflat)
        g16 = g.reshape(16, R, table.shape[1])
        table = _tc_layer(g16, pos8, Wf, Wp, b, acts[l])
    return table


def kernel(pos, feat, enc_W, enc_b, Wg, bg, Egrid, dec_W, dec_b, dec_Ws,
           dec_bs):
    B, N, _ = pos.shape

    def jmaj(idx, R):
        return jnp.transpose(idx.reshape(R, 16)).reshape(-1)

    pos8 = _pad_cols(pos, 8)
    idx = _knn_pallas(pos8)
    R1 = B * N
    pos8f = pos8.reshape(R1, 8)
    table1 = jnp.concatenate(
        [_pad_cols(pos8f, 16), _pad_cols(feat.reshape(R1, 1), 16)], axis=1)
    t1 = _stage(table1, pos8f, jmaj(idx, R1), enc_W[0:3], enc_b[0:3],
                ["relu", "relu", "relu"], R1)

    m1 = N // 4
    posT = jnp.transpose(pos8, (0, 2, 1))
    s1 = _fps_pallas_call(posT, m1)[:, :, 0]
    s1g = (s1 + (jnp.arange(B, dtype=_I32) * N)[:, None]).reshape(-1)
    R2 = B * m1
    rows2 = _gather_rows(t1, s1g)
    pos8_2 = rows2[:, :8]
    pos2 = pos8_2[:, :3].reshape(B, m1, 3)

    idx2 = _knn_pallas(_pad_cols(pos2, 8))
    t2 = _stage(rows2, pos8_2, jmaj(idx2, R2), enc_W[3:7], enc_b[3:7],
                ["relu"] * 4, R2)

    m2 = m1 // 16
    posT2 = jnp.transpose(pos2, (0, 2, 1))
    posT2 = jnp.pad(posT2, ((0, 0), (0, 5), (0, 0)))
    s2 = _fps_pallas_call(posT2, m2)[:, :, 0]
    s2g = (s2 + (jnp.arange(B, dtype=_I32) * m1)[:, None]).reshape(-1)
    R3 = B * m2
    rows3 = _gather_rows(t2, s2g)
    pos8_3 = rows3[:, :8]
    pos3 = pos8_3[:, :3].reshape(B, m2, 3)

    idx3 = _knn_pallas(_pad_cols(pos3, 8))
    t3 = _stage(rows3, pos8_3, jmaj(idx3, R3), enc_W[7:14], enc_b[7:14],
                ["relu"] * 6 + ["none"], R3)
    f3 = t3[:, 16:]

    latent_pos = pos3
    latent_feat = f3[:, :13].reshape(B, m2, 13)

    featp = _pad_cols(f3[:, :13], 16).reshape(B, m2, 16)
    cat = jnp.concatenate([latent_feat, latent_pos], axis=2)
    Wg16 = _pad_rows(Wg, 16)
    dW = [dec_W[0], dec_W[1], _pad_cols(dec_W[2], 8)]
    db = [dec_b[0][None, :], dec_b[1][None, :],
          _pad_cols(dec_b[2][None, :], 8)]
    dWs = [dec_Ws[0], dec_Ws[1], _pad_cols(dec_Ws[2].reshape(64, 2, 3),
                                           8).reshape(64, 16)]
    dbs = [dec_bs[0][None, :], dec_bs[1][None, :],
          _pad_cols(dec_bs[2].reshape(1, 2, 3), 8).reshape(1, 16)]
    dec = _decoder(cat, featp, Wg16, bg[None, :], Egrid, dW, db, dWs, dbs)
    dec = dec[:, :, :3]
    return (latent_pos, latent_feat, dec)
